# baseline (speedup 1.0000x reference)
import functools

import jax
import jax.numpy as jnp
from jax import lax
from jax.experimental import pallas as pl
from jax.experimental.pallas import tpu as pltpu
from jax.experimental.pallas import tpu_sc as plsc

NC = 2
NS = 16
NW = NC * NS
L = 16

B = 16384
D = 64
BPW = B // NW
CHUNK = 128
NCHUNK = BPW // CHUNK
GROUPS = BPW // L


@functools.partial(
    pl.kernel,
    out_type=jax.ShapeDtypeStruct((B,), jnp.float32),
    mesh=plsc.VectorSubcoreMesh(core_axis_name="c", subcore_axis_name="s"),
    scratch_types=[
        pltpu.VMEM((NCHUNK, CHUNK), jnp.int32),
        pltpu.VMEM((NCHUNK, CHUNK), jnp.int32),
        pltpu.VMEM((BPW, D), jnp.float32),
        pltpu.VMEM((BPW, D), jnp.float32),
        pltpu.VMEM((BPW,), jnp.float32),
        pltpu.SemaphoreType.DMA,
        pltpu.SemaphoreType.DMA,
    ],
    compiler_params=pltpu.CompilerParams(
        needs_layout_passes=False, use_tc_tiling_on_sc=False),
)
def _dot_gather(user_hbm, item_hbm, uf_hbm, if_hbm, out_hbm,
                idx_u, idx_i, rows_u, rows_i, out_v, sem_u, sem_i):
    wid = lax.axis_index("s") * NC + lax.axis_index("c")
    base = wid * BPW

    pltpu.sync_copy(user_hbm.at[wid], idx_u)
    pltpu.sync_copy(item_hbm.at[wid], idx_i)

    copies = []
    for j in range(NCHUNK):
        cu = pltpu.async_copy(uf_hbm.at[idx_u.at[j]],
                              rows_u.at[pl.ds(j * CHUNK, CHUNK)], sem_u)
        ci = pltpu.async_copy(if_hbm.at[idx_i.at[j]],
                              rows_i.at[pl.ds(j * CHUNK, CHUNK)], sem_i)
        copies.append((cu, ci))
    for cu, ci in copies:
        cu.wait()
        ci.wait()

    iota = lax.iota(jnp.int32, L)
    zeros_i = jnp.zeros((L,), jnp.int32)

    def group_body(g, carry):
        rows = g * L + iota

        def col_body(c, acc):
            cols = zeros_i + c
            u16 = plsc.load_gather(rows_u, [rows, cols])
            v16 = plsc.load_gather(rows_i, [rows, cols])
            return acc + u16 * v16

        acc = lax.fori_loop(0, D, col_body, jnp.zeros((L,), jnp.float32))
        out_v[pl.ds(g * L, L)] = acc
        return carry

    lax.fori_loop(0, GROUPS, group_body, 0)

    pltpu.sync_copy(out_v, out_hbm.at[pl.ds(base, BPW)])


def kernel(user, item, user_factors, item_factors):
    user2 = user.astype(jnp.int32).reshape(NW, NCHUNK, CHUNK)
    item2 = item.astype(jnp.int32).reshape(NW, NCHUNK, CHUNK)
    return _dot_gather(user2, item2, user_factors, item_factors)
